# baseline (speedup 1.0000x reference)
import jax
import jax.numpy as jnp
from jax.experimental import pallas as pl
from jax.experimental.pallas import tpu as pltpu

_C = 512


def _fused_kernel(len_ref, x_ref, w1_ref, b1_ref, w2_ref,
                  pa_ref, alp_ref, ent_ref, sv_ref, prev_ref):
    b = pl.program_id(0)
    i = pl.program_id(1)
    nc_max = pl.num_programs(1)
    L = nc_max * _C
    D = x_ref.shape[2]
    nv = len_ref[b] - 1
    nch = nv // _C + 1

    @pl.when(i < nch)
    def _compute():
        xb = x_ref[0]
        xa = jnp.concatenate([prev_ref[...], xb[:-1, :]], axis=0)
        prev_ref[...] = xb[-1:, :]
        h = jnp.maximum(
            jnp.dot(xa, w1_ref[:D, :], preferred_element_type=jnp.float32)
            + jnp.dot(xb, w1_ref[D:, :], preferred_element_type=jnp.float32)
            + b1_ref[...], 0.0)
        s = jnp.sum(h * w2_ref[...], axis=1, keepdims=True)
        lane_i = jax.lax.broadcasted_iota(jnp.int32, (1, nc_max), 1)
        sv_ref[...] = jnp.where(lane_i == i, s, sv_ref[...])

    @pl.when(i == nch - 1)
    def _finalize():
        lane_i = jax.lax.broadcasted_iota(jnp.int32, (1, nc_max), 1)
        t_mat = (jax.lax.broadcasted_iota(jnp.int32, (_C, nc_max), 0)
                 + lane_i * _C - 1)
        neg_big = jnp.float32(-1e30)
        valid = (t_mat >= 0) & (t_mat < nv)
        s_m = jnp.where(valid, sv_ref[...], neg_big)
        m = jnp.max(s_m)
        sm = jnp.where(valid, s_m - m, 0.0)
        e = jnp.where(valid, jnp.exp(sm), 0.0)
        z = jnp.sum(e)
        t = jnp.sum(e * sm)
        logz = jnp.log(z)
        pa_ref[b] = jnp.min(jnp.where(s_m == m, t_mat, L))
        alp_ref[b] = -logz
        ent_ref[b] = logz - t / z


def kernel(sequence_embedding, sentence_lengths, W1, b1, W2, b2):
    x = sequence_embedding
    B, L, D = x.shape
    H = W1.shape[1]
    nc_max = L // _C

    def idx_x(b, i, len_ref):
        nch = (len_ref[b] - 1) // _C + 1
        return (b, jnp.minimum(i, nch - 1), 0)

    grid_spec = pltpu.PrefetchScalarGridSpec(
        num_scalar_prefetch=1,
        grid=(B, nc_max),
        in_specs=[
            pl.BlockSpec((1, _C, D), idx_x),
            pl.BlockSpec((2 * D, H), lambda b, i, *_: (0, 0)),
            pl.BlockSpec((1, H), lambda b, i, *_: (0, 0)),
            pl.BlockSpec((1, H), lambda b, i, *_: (0, 0)),
        ],
        out_specs=(
            pl.BlockSpec((B,), lambda b, i, *_: (0,),
                         memory_space=pltpu.SMEM),
            pl.BlockSpec((B,), lambda b, i, *_: (0,),
                         memory_space=pltpu.SMEM),
            pl.BlockSpec((B,), lambda b, i, *_: (0,),
                         memory_space=pltpu.SMEM),
        ),
        scratch_shapes=[
            pltpu.VMEM((_C, nc_max), jnp.float32),
            pltpu.VMEM((1, D), jnp.float32),
        ],
    )
    pa, alp, ent = pl.pallas_call(
        _fused_kernel,
        grid_spec=grid_spec,
        out_shape=(
            jax.ShapeDtypeStruct((B,), jnp.int32),
            jax.ShapeDtypeStruct((B,), jnp.float32),
            jax.ShapeDtypeStruct((B,), jnp.float32),
        ),
        compiler_params=pltpu.CompilerParams(
            dimension_semantics=("arbitrary", "arbitrary"),
        ),
    )(sentence_lengths, x, W1, b1.reshape(1, H), W2.reshape(1, H))
    return (pa, alp, ent)


# speedup vs baseline: 1.4194x; 1.4194x over previous
import jax
import jax.numpy as jnp
from jax.experimental import pallas as pl
from jax.experimental.pallas import tpu as pltpu

_C = 512


def _fused_kernel(len_ref, x_ref, w1_ref, b1_ref, w2_ref,
                  pa_ref, alp_ref, ent_ref):
    b = pl.program_id(0)
    L = x_ref.shape[1]
    D = x_ref.shape[2]
    nv = len_ref[b] - 1

    w1a = w1_ref[:D, :]
    w1b = w1_ref[D:, :]
    b1 = b1_ref[...]
    w2 = w2_ref[...]

    nchunks = (nv + _C - 1) // _C
    nc_max = L // _C
    neg_big = jnp.float32(-1e30)
    lane_i = jax.lax.broadcasted_iota(jnp.int32, (1, nc_max), 1)

    def body(i, sv):
        base = i * _C
        xa = x_ref[0, pl.ds(base, _C), :]
        xlast = x_ref[0, pl.ds(jnp.minimum(base + _C, L - 1), 1), :]
        xb = jnp.concatenate([xa[1:, :], xlast], axis=0)
        h = jnp.maximum(
            jnp.dot(xa, w1a, preferred_element_type=jnp.float32)
            + jnp.dot(xb, w1b, preferred_element_type=jnp.float32)
            + b1, 0.0)
        s = jnp.sum(h * w2, axis=1, keepdims=True)
        return jnp.where(lane_i == i, s, sv)

    sv = jax.lax.fori_loop(
        0, nchunks, body,
        jnp.full((_C, nc_max), neg_big, jnp.float32))

    t_mat = (jax.lax.broadcasted_iota(jnp.int32, (_C, nc_max), 0)
             + lane_i * _C)
    valid = t_mat < nv
    s_m = jnp.where(valid, sv, neg_big)
    m = jnp.max(s_m)
    sm = jnp.where(valid, s_m - m, 0.0)
    e = jnp.where(valid, jnp.exp(sm), 0.0)
    z = jnp.sum(e)
    t = jnp.sum(e * sm)
    logz = jnp.log(z)
    pa_ref[b] = jnp.min(jnp.where(s_m == m, t_mat, L))
    alp_ref[b] = -logz
    ent_ref[b] = logz - t / z


def kernel(sequence_embedding, sentence_lengths, W1, b1, W2, b2):
    x = sequence_embedding
    B, L, D = x.shape
    H = W1.shape[1]

    grid_spec = pltpu.PrefetchScalarGridSpec(
        num_scalar_prefetch=1,
        grid=(B,),
        in_specs=[
            pl.BlockSpec((1, L, D), lambda b, *_: (b, 0, 0)),
            pl.BlockSpec((2 * D, H), lambda b, *_: (0, 0)),
            pl.BlockSpec((1, H), lambda b, *_: (0, 0)),
            pl.BlockSpec((1, H), lambda b, *_: (0, 0)),
        ],
        out_specs=(
            pl.BlockSpec((B,), lambda b, *_: (0,), memory_space=pltpu.SMEM),
            pl.BlockSpec((B,), lambda b, *_: (0,), memory_space=pltpu.SMEM),
            pl.BlockSpec((B,), lambda b, *_: (0,), memory_space=pltpu.SMEM),
        ),
    )
    pa, alp, ent = pl.pallas_call(
        _fused_kernel,
        grid_spec=grid_spec,
        out_shape=(
            jax.ShapeDtypeStruct((B,), jnp.int32),
            jax.ShapeDtypeStruct((B,), jnp.float32),
            jax.ShapeDtypeStruct((B,), jnp.float32),
        ),
        compiler_params=pltpu.CompilerParams(
            dimension_semantics=("arbitrary",),
        ),
    )(sentence_lengths, x, W1, b1.reshape(1, H), W2.reshape(1, H))
    return (pa, alp, ent)
